# baseline (speedup 1.0000x reference)
import functools

import jax
import jax.numpy as jnp
from jax import lax
from jax.experimental import pallas as pl
from jax.experimental.pallas import tpu as pltpu
from jax.experimental.pallas import tpu_sc as plsc

N_NODES = 10000
D = 128
NC = 2
NS = 16
NW = NC * NS
N_PAD = 10240
RPT = N_PAD // NS
CH = 128
IDX_G = 16
GROUPS = 5
CHUNKS = GROUPS * IDX_G
E_TILE = CHUNKS * CH
E_PAD = NW * E_TILE
MBLK = 2048
NCHUNK = E_PAD // CH
PBLK = NCHUNK // 5

_mesh = plsc.VectorSubcoreMesh(
    core_axis_name="c", subcore_axis_name="s", num_cores=NC, num_subcores=NS
)


@functools.partial(
    pl.kernel,
    out_type=jax.ShapeDtypeStruct((NC, N_PAD), jnp.float32),
    mesh=_mesh,
    scratch_types=[
        pltpu.VMEM((CHUNKS, CH), jnp.int32),
        pltpu.VMEM((CH,), jnp.float32),
        pltpu.VMEM((RPT,), jnp.float32),
        pltpu.SemaphoreType.DMA,
        pltpu.VMEM_SHARED((N_PAD,), jnp.float32),
    ],
)
def _deg_kernel(dst_hbm, out_hbm, idx_v, ones_v, zeros_v, dsem, deg_sh):
    c = lax.axis_index("c")
    s = lax.axis_index("s")
    wid = s * NC + c

    @pl.loop(0, CH // 16)
    def _(i):
        ones_v[pl.ds(i * 16, 16)] = jnp.ones((16,), jnp.float32)

    @pl.loop(0, RPT // 16)
    def _(i):
        zeros_v[pl.ds(i * 16, 16)] = jnp.zeros((16,), jnp.float32)

    pltpu.sync_copy(zeros_v, deg_sh.at[pl.ds(s * RPT, RPT)])
    pltpu.sync_copy(dst_hbm.at[pl.ds(wid * CHUNKS, CHUNKS)], idx_v)
    plsc.subcore_barrier()

    @pl.loop(0, GROUPS)
    def _(grp):
        for k in range(IDX_G):
            pltpu.make_async_copy(
                ones_v, deg_sh.at[idx_v.at[grp * IDX_G + k]], dsem
            ).start(add=True)
        for k in range(IDX_G):
            pltpu.make_async_copy(
                ones_v, deg_sh.at[idx_v.at[grp * IDX_G + k]], dsem
            ).wait()

    plsc.subcore_barrier()
    pltpu.sync_copy(
        deg_sh.at[pl.ds(s * RPT, RPT)], out_hbm.at[c, pl.ds(s * RPT, RPT)]
    )


@functools.partial(
    pl.kernel,
    out_type=jax.ShapeDtypeStruct((NC, N_PAD, D), jnp.float32),
    mesh=_mesh,
    scratch_types=[
        pltpu.VMEM((IDX_G, CH), jnp.int32),
        pltpu.VMEM((IDX_G, CH), jnp.int32),
        pltpu.VMEM((IDX_G, CH), jnp.int32),
        pltpu.VMEM((IDX_G, CH), jnp.int32),
        pltpu.VMEM((CH, D), jnp.float32),
        pltpu.VMEM((CH, D), jnp.float32),
        pltpu.SemaphoreType.DMA,
        pltpu.SemaphoreType.DMA,
        pltpu.SemaphoreType.DMA,
        pltpu.SemaphoreType.DMA,
        pltpu.SemaphoreType.DMA,
        pltpu.SemaphoreType.DMA,
        pltpu.VMEM_SHARED((N_PAD, D), jnp.float32),
    ],
)
def _edge_kernel(g_hbm, src_hbm, dst_hbm, out_hbm,
                 srcI0, srcI1, dstI0, dstI1, buf0, buf1,
                 gsem0, gsem1, ssem0, ssem1, isem_s, isem_d, acc_sh):
    c = lax.axis_index("c")
    s = lax.axis_index("s")
    wid = s * NC + c
    srcIs = (srcI0, srcI1)
    dstIs = (dstI0, dstI1)
    bufs = (buf0, buf1)
    gsems = (gsem0, gsem1)
    ssems = (ssem0, ssem1)

    @pl.loop(0, CH)
    def _(i):
        for j in range(D // 16):
            z = jnp.zeros((16,), jnp.float32)
            bufs[0][i, pl.ds(j * 16, 16)] = z
            bufs[1][i, pl.ds(j * 16, 16)] = z

    @pl.loop(0, RPT // CH)
    def _(k):
        pltpu.sync_copy(bufs[0], acc_sh.at[pl.ds(s * RPT + k * CH, CH)])

    base = wid * CHUNKS
    pltpu.sync_copy(src_hbm.at[pl.ds(base, IDX_G)], srcIs[0])
    pltpu.sync_copy(dst_hbm.at[pl.ds(base, IDX_G)], dstIs[0])

    pltpu.async_copy(bufs[1], acc_sh.at[pl.ds(s * RPT, CH)], ssems[1])
    pltpu.async_copy(g_hbm.at[srcIs[0].at[0]], bufs[0], gsems[0])
    plsc.subcore_barrier()

    for grp in range(GROUPS):
        sI = srcIs[grp % 2]
        dI = dstIs[grp % 2]
        if grp + 1 < GROUPS:
            nbase = base + (grp + 1) * IDX_G
            pltpu.async_copy(src_hbm.at[pl.ds(nbase, IDX_G)],
                             srcIs[(grp + 1) % 2], isem_s)
            pltpu.async_copy(dst_hbm.at[pl.ds(nbase, IDX_G)],
                             dstIs[(grp + 1) % 2], isem_d)

        @pl.loop(0, IDX_G, step=2)
        def _(t):
            for k in range(2):
                j = t + k
                pltpu.make_async_copy(
                    g_hbm.at[sI.at[j]], bufs[k], gsems[k]
                ).wait()
                pltpu.make_async_copy(
                    bufs[1 - k], acc_sh.at[dI.at[j]], ssems[1 - k]
                ).wait()
                pltpu.make_async_copy(
                    bufs[k], acc_sh.at[dI.at[j]], ssems[k]
                ).start(add=True)

                @pl.when(j + 1 < IDX_G)
                def _():
                    pltpu.async_copy(
                        g_hbm.at[sI.at[j + 1]], bufs[1 - k], gsems[1 - k]
                    )

        if grp + 1 < GROUPS:
            nbase = base + (grp + 1) * IDX_G
            nsI = srcIs[(grp + 1) % 2]
            pltpu.make_async_copy(src_hbm.at[pl.ds(nbase, IDX_G)],
                                  nsI, isem_s).wait()
            pltpu.make_async_copy(dst_hbm.at[pl.ds(nbase, IDX_G)],
                                  dstIs[(grp + 1) % 2], isem_d).wait()
            pltpu.async_copy(g_hbm.at[nsI.at[0]], bufs[0], gsems[0])

    pltpu.make_async_copy(
        bufs[1], acc_sh.at[dstIs[(GROUPS - 1) % 2].at[IDX_G - 1]], ssems[1]
    ).wait()
    plsc.subcore_barrier()
    pltpu.sync_copy(
        acc_sh.at[pl.ds(s * RPT, RPT)], out_hbm.at[c, pl.ds(s * RPT, RPT)]
    )


def _prep_body(e_ref, s_ref, d_ref):
    i = pl.program_id(0)
    e = e_ref[...]
    rows = lax.broadcasted_iota(jnp.int32, (PBLK, CH), 0) + i * PBLK
    pad = N_NODES + lax.broadcasted_iota(jnp.int32, (PBLK, CH), 1)
    mask = rows < (320000 // CH)
    s_ref[...] = jnp.where(mask, e[0].reshape(PBLK, CH), pad)
    d_ref[...] = jnp.where(mask, e[1].reshape(PBLK, CH), pad)


_prep = pl.pallas_call(
    _prep_body,
    grid=(NCHUNK // PBLK,),
    in_specs=[pl.BlockSpec((2, PBLK * CH), lambda i: (0, i))],
    out_specs=[
        pl.BlockSpec((PBLK, CH), lambda i: (i, 0)),
        pl.BlockSpec((PBLK, CH), lambda i: (i, 0)),
    ],
    out_shape=[
        jax.ShapeDtypeStruct((NCHUNK, CH), jnp.int32),
        jax.ShapeDtypeStruct((NCHUNK, CH), jnp.int32),
    ],
)


def _mm_body(x_ref, w_ref, degp_ref, g_ref):
    deg = degp_ref[0] + degp_ref[1] + 1.0
    dinv = lax.rsqrt(deg).reshape(MBLK, 1)
    h = jnp.dot(x_ref[...], w_ref[...], preferred_element_type=jnp.float32)
    g_ref[...] = h * dinv


_mm = pl.pallas_call(
    _mm_body,
    grid=(N_PAD // MBLK,),
    in_specs=[
        pl.BlockSpec((MBLK, D), lambda i: (i, 0)),
        pl.BlockSpec((D, D), lambda i: (0, 0)),
        pl.BlockSpec((2, MBLK), lambda i: (0, i)),
    ],
    out_specs=pl.BlockSpec((MBLK, D), lambda i: (i, 0)),
    out_shape=jax.ShapeDtypeStruct((N_PAD, D), jnp.float32),
)


def _fin_body(p_ref, g_ref, degp_ref, b_ref, fcw_ref, fcb_ref, y_ref):
    deg = degp_ref[0] + degp_ref[1] + 1.0
    dinv = lax.rsqrt(deg).reshape(MBLK, 1)
    h = (p_ref[0] + p_ref[1] + g_ref[...]) * dinv + b_ref[...]
    h = jnp.maximum(h, 0.0)
    y = jnp.sum(h * fcw_ref[...], axis=1, keepdims=True) + fcb_ref[...]
    y_ref[...] = jnp.maximum(y, 0.0)


_fin = pl.pallas_call(
    _fin_body,
    grid=(N_PAD // MBLK,),
    in_specs=[
        pl.BlockSpec((2, MBLK, D), lambda i: (0, i, 0)),
        pl.BlockSpec((MBLK, D), lambda i: (i, 0)),
        pl.BlockSpec((2, MBLK), lambda i: (0, i)),
        pl.BlockSpec((1, D), lambda i: (0, 0)),
        pl.BlockSpec((1, D), lambda i: (0, 0)),
        pl.BlockSpec((1, 1), lambda i: (0, 0)),
    ],
    out_specs=pl.BlockSpec((MBLK, 1), lambda i: (i, 0)),
    out_shape=jax.ShapeDtypeStruct((N_NODES, 1), jnp.float32),
)


def kernel(x, edge_index, temporal_features, W, b, fc_W, fc_b):
    del temporal_features
    srcp, dstp = _prep(edge_index.astype(jnp.int32))

    degp = _deg_kernel(dstp)

    g = _mm(x, W, degp)

    parts = _edge_kernel(g, srcp, dstp)

    return _fin(parts, g, degp, b.reshape(1, D), fc_W.reshape(1, D),
                fc_b.reshape(1, 1))


# speedup vs baseline: 1.1130x; 1.1130x over previous
import functools

import jax
import jax.numpy as jnp
from jax import lax
from jax.experimental import pallas as pl
from jax.experimental.pallas import tpu as pltpu
from jax.experimental.pallas import tpu_sc as plsc

N_NODES = 10000
D = 128
NC = 2
NS = 16
NW = NC * NS
N_PAD = 10240
RPT = N_PAD // NS
CH = 128
IDX_G = 16
GROUPS = 5
CHUNKS = GROUPS * IDX_G
E_TILE = CHUNKS * CH
E_PAD = NW * E_TILE
MBLK = 2048
NCHUNK = E_PAD // CH
PBLK = NCHUNK // 5

_mesh = plsc.VectorSubcoreMesh(
    core_axis_name="c", subcore_axis_name="s", num_cores=NC, num_subcores=NS
)


@functools.partial(
    pl.kernel,
    out_type=jax.ShapeDtypeStruct((NC, N_PAD), jnp.float32),
    mesh=_mesh,
    scratch_types=[
        pltpu.VMEM((CHUNKS, CH), jnp.int32),
        pltpu.VMEM((CH,), jnp.float32),
        pltpu.VMEM((RPT,), jnp.float32),
        pltpu.SemaphoreType.DMA,
        pltpu.VMEM_SHARED((N_PAD,), jnp.float32),
    ],
)
def _deg_kernel(dst_hbm, out_hbm, idx_v, ones_v, zeros_v, dsem, deg_sh):
    c = lax.axis_index("c")
    s = lax.axis_index("s")
    wid = s * NC + c

    @pl.loop(0, CH // 16)
    def _(i):
        ones_v[pl.ds(i * 16, 16)] = jnp.ones((16,), jnp.float32)

    @pl.loop(0, RPT // 16)
    def _(i):
        zeros_v[pl.ds(i * 16, 16)] = jnp.zeros((16,), jnp.float32)

    pltpu.sync_copy(zeros_v, deg_sh.at[pl.ds(s * RPT, RPT)])
    pltpu.sync_copy(dst_hbm.at[pl.ds(wid * CHUNKS, CHUNKS)], idx_v)
    plsc.subcore_barrier()

    @pl.loop(0, GROUPS)
    def _(grp):
        for k in range(IDX_G):
            pltpu.make_async_copy(
                ones_v, deg_sh.at[idx_v.at[grp * IDX_G + k]], dsem
            ).start(add=True)
        for k in range(IDX_G):
            pltpu.make_async_copy(
                ones_v, deg_sh.at[idx_v.at[grp * IDX_G + k]], dsem
            ).wait()

    plsc.subcore_barrier()
    pltpu.sync_copy(
        deg_sh.at[pl.ds(s * RPT, RPT)], out_hbm.at[c, pl.ds(s * RPT, RPT)]
    )


@functools.partial(
    pl.kernel,
    out_type=jax.ShapeDtypeStruct((NC, N_PAD, D), jnp.float32),
    mesh=_mesh,
    scratch_types=[
        pltpu.VMEM((IDX_G, CH), jnp.int32),
        pltpu.VMEM((IDX_G, CH), jnp.int32),
        pltpu.VMEM((IDX_G, CH), jnp.int32),
        pltpu.VMEM((IDX_G, CH), jnp.int32),
        pltpu.VMEM((CH, D), jnp.float32),
        pltpu.VMEM((CH, D), jnp.float32),
        pltpu.SemaphoreType.DMA,
        pltpu.SemaphoreType.DMA,
        pltpu.SemaphoreType.DMA,
        pltpu.SemaphoreType.DMA,
        pltpu.SemaphoreType.DMA,
        pltpu.SemaphoreType.DMA,
        pltpu.VMEM_SHARED((N_PAD, D), jnp.float32),
    ],
)
def _edge_kernel(g_hbm, src_hbm, dst_hbm, out_hbm,
                 srcI0, srcI1, dstI0, dstI1, buf0, buf1,
                 gsem0, gsem1, ssem0, ssem1, isem_s, isem_d, acc_sh):
    c = lax.axis_index("c")
    s = lax.axis_index("s")
    wid = s * NC + c
    srcIs = (srcI0, srcI1)
    dstIs = (dstI0, dstI1)
    bufs = (buf0, buf1)
    gsems = (gsem0, gsem1)
    ssems = (ssem0, ssem1)

    @pl.loop(0, CH)
    def _(i):
        for j in range(D // 16):
            bufs[0][i, pl.ds(j * 16, 16)] = jnp.zeros((16,), jnp.float32)

    @pl.loop(0, RPT // CH)
    def _(k):
        pltpu.sync_copy(bufs[0], acc_sh.at[pl.ds(s * RPT + k * CH, CH)])

    base = wid * CHUNKS
    pltpu.sync_copy(src_hbm.at[pl.ds(base, IDX_G)], srcIs[0])
    pltpu.sync_copy(dst_hbm.at[pl.ds(base, IDX_G)], dstIs[0])
    plsc.subcore_barrier()

    for k in range(2):
        pltpu.async_copy(g_hbm.at[srcIs[0].at[k]], bufs[k], gsems[k])

    for grp in range(GROUPS):
        sI = srcIs[grp % 2]
        dI = dstIs[grp % 2]
        if grp + 1 < GROUPS:
            nbase = base + (grp + 1) * IDX_G
            pltpu.async_copy(src_hbm.at[pl.ds(nbase, IDX_G)],
                             srcIs[(grp + 1) % 2], isem_s)
            pltpu.async_copy(dst_hbm.at[pl.ds(nbase, IDX_G)],
                             dstIs[(grp + 1) % 2], isem_d)

        @pl.loop(0, IDX_G, step=2)
        def _(t):
            for k in range(2):
                pltpu.make_async_copy(
                    g_hbm.at[sI.at[t + k]], bufs[k], gsems[k]
                ).wait()
                pltpu.sync_copy(bufs[k], acc_sh.at[dI.at[t + k]], add=True)

                @pl.when(t + k + 2 < IDX_G)
                def _():
                    pltpu.async_copy(
                        g_hbm.at[sI.at[t + k + 2]], bufs[k], gsems[k]
                    )

        if grp + 1 < GROUPS:
            nbase = base + (grp + 1) * IDX_G
            nsI = srcIs[(grp + 1) % 2]
            pltpu.make_async_copy(src_hbm.at[pl.ds(nbase, IDX_G)],
                                  nsI, isem_s).wait()
            pltpu.make_async_copy(dst_hbm.at[pl.ds(nbase, IDX_G)],
                                  dstIs[(grp + 1) % 2], isem_d).wait()
            for k in range(2):
                pltpu.async_copy(g_hbm.at[nsI.at[k]], bufs[k], gsems[k])

    plsc.subcore_barrier()
    pltpu.sync_copy(
        acc_sh.at[pl.ds(s * RPT, RPT)], out_hbm.at[c, pl.ds(s * RPT, RPT)]
    )


def _prep_body(e_ref, s_ref, d_ref):
    i = pl.program_id(0)
    e = e_ref[...]
    rows = lax.broadcasted_iota(jnp.int32, (PBLK, CH), 0) + i * PBLK
    pad = N_NODES + lax.broadcasted_iota(jnp.int32, (PBLK, CH), 1)
    mask = rows < (320000 // CH)
    s_ref[...] = jnp.where(mask, e[0].reshape(PBLK, CH), pad)
    d_ref[...] = jnp.where(mask, e[1].reshape(PBLK, CH), pad)


_prep = pl.pallas_call(
    _prep_body,
    grid=(NCHUNK // PBLK,),
    in_specs=[pl.BlockSpec((2, PBLK * CH), lambda i: (0, i))],
    out_specs=[
        pl.BlockSpec((PBLK, CH), lambda i: (i, 0)),
        pl.BlockSpec((PBLK, CH), lambda i: (i, 0)),
    ],
    out_shape=[
        jax.ShapeDtypeStruct((NCHUNK, CH), jnp.int32),
        jax.ShapeDtypeStruct((NCHUNK, CH), jnp.int32),
    ],
)


def _mm_body(x_ref, w_ref, degp_ref, g_ref):
    deg = degp_ref[0] + degp_ref[1] + 1.0
    dinv = lax.rsqrt(deg).reshape(MBLK, 1)
    h = jnp.dot(x_ref[...], w_ref[...], preferred_element_type=jnp.float32)
    g_ref[...] = h * dinv


_mm = pl.pallas_call(
    _mm_body,
    grid=(N_PAD // MBLK,),
    in_specs=[
        pl.BlockSpec((MBLK, D), lambda i: (i, 0)),
        pl.BlockSpec((D, D), lambda i: (0, 0)),
        pl.BlockSpec((2, MBLK), lambda i: (0, i)),
    ],
    out_specs=pl.BlockSpec((MBLK, D), lambda i: (i, 0)),
    out_shape=jax.ShapeDtypeStruct((N_PAD, D), jnp.float32),
)


def _fin_body(p_ref, g_ref, degp_ref, b_ref, fcw_ref, fcb_ref, y_ref):
    deg = degp_ref[0] + degp_ref[1] + 1.0
    dinv = lax.rsqrt(deg).reshape(MBLK, 1)
    h = (p_ref[0] + p_ref[1] + g_ref[...]) * dinv + b_ref[...]
    h = jnp.maximum(h, 0.0)
    y = jnp.sum(h * fcw_ref[...], axis=1, keepdims=True) + fcb_ref[...]
    y_ref[...] = jnp.maximum(y, 0.0)


_fin = pl.pallas_call(
    _fin_body,
    grid=(N_PAD // MBLK,),
    in_specs=[
        pl.BlockSpec((2, MBLK, D), lambda i: (0, i, 0)),
        pl.BlockSpec((MBLK, D), lambda i: (i, 0)),
        pl.BlockSpec((2, MBLK), lambda i: (0, i)),
        pl.BlockSpec((1, D), lambda i: (0, 0)),
        pl.BlockSpec((1, D), lambda i: (0, 0)),
        pl.BlockSpec((1, 1), lambda i: (0, 0)),
    ],
    out_specs=pl.BlockSpec((MBLK, 1), lambda i: (i, 0)),
    out_shape=jax.ShapeDtypeStruct((N_NODES, 1), jnp.float32),
)


def kernel(x, edge_index, temporal_features, W, b, fc_W, fc_b):
    del temporal_features
    srcp, dstp = _prep(edge_index.astype(jnp.int32))

    degp = _deg_kernel(dstp)

    g = _mm(x, W, degp)

    parts = _edge_kernel(g, srcp, dstp)

    return _fin(parts, g, degp, b.reshape(1, D), fc_W.reshape(1, D),
                fc_b.reshape(1, 1))
